# baseline (speedup 1.0000x reference)
import functools

import jax
import jax.numpy as jnp
from jax import lax
from jax.experimental import pallas as pl
from jax.experimental.pallas import tpu as pltpu
from jax.experimental.pallas import tpu_sc as plsc

BATCH = 16384
D = 64
NC = 2
NS = 16
NW = NC * NS
B_PER_W = BATCH // NW
CHUNK = 128
N_CHUNKS = B_PER_W // CHUNK


def _sc_gather_body(u_hbm, i_hbm, j_hbm, utab_hbm, itab_hbm,
                    ue_hbm, ie_hbm, je_hbm,
                    idx_u, idx_i, idx_j, sem):
    wid = lax.axis_index("s") * NC + lax.axis_index("c")
    base = wid * B_PER_W
    pltpu.sync_copy(u_hbm.at[pl.ds(base, B_PER_W)], idx_u)
    pltpu.sync_copy(i_hbm.at[pl.ds(base, B_PER_W)], idx_i)
    pltpu.sync_copy(j_hbm.at[pl.ds(base, B_PER_W)], idx_j)

    @pl.loop(0, B_PER_W // 16)
    def _grp(g):
        k0 = g * 16
        v_u = idx_u[pl.ds(k0, 16)]
        v_i = idx_i[pl.ds(k0, 16)]
        v_j = idx_j[pl.ds(k0, 16)]
        for l in range(16):
            pltpu.async_copy(utab_hbm.at[pl.ds(v_u[l], 1)],
                             ue_hbm.at[pl.ds(base + k0 + l, 1)], sem)
            pltpu.async_copy(itab_hbm.at[pl.ds(v_i[l], 1)],
                             ie_hbm.at[pl.ds(base + k0 + l, 1)], sem)
            pltpu.async_copy(itab_hbm.at[pl.ds(v_j[l], 1)],
                             je_hbm.at[pl.ds(base + k0 + l, 1)], sem)

    pltpu.make_async_copy(utab_hbm.at[pl.ds(0, 3 * B_PER_W)],
                          ue_hbm.at[pl.ds(0, 3 * B_PER_W)], sem).wait()


@jax.jit
def _sc_gather(u, i, j, utab, itab):
    mesh = plsc.VectorSubcoreMesh(core_axis_name="c", subcore_axis_name="s",
                                  num_cores=NC, num_subcores=NS)
    emb = jax.ShapeDtypeStruct((BATCH, D), jnp.float32)
    return pl.kernel(
        _sc_gather_body,
        out_type=(emb, emb, emb),
        mesh=mesh,
        scratch_types=[
            pltpu.VMEM((B_PER_W,), jnp.int32),
            pltpu.VMEM((B_PER_W,), jnp.int32),
            pltpu.VMEM((B_PER_W,), jnp.int32),
            pltpu.SemaphoreType.DMA,
        ],
    )(u, i, j, utab, itab)


def _mlp_body(ue_ref, ie_ref, je_ref, w1_ref, b1_ref, w2_ref, b2_ref,
              si_ref, sj_ref):
    ue = ue_ref[...]
    h = jnp.dot(ue, w1_ref[...].T, preferred_element_type=jnp.float32)
    h = jnp.maximum(h + b1_ref[...], 0.0)
    h = jnp.dot(h, w2_ref[...].T, preferred_element_type=jnp.float32)
    h = jnp.maximum(h + b2_ref[...], 0.0)
    si_ref[...] = jnp.sum(h * ie_ref[...], axis=1, keepdims=True)
    sj_ref[...] = jnp.sum(h * je_ref[...], axis=1, keepdims=True)


@jax.jit
def _tc_mlp(ue, ie, je, W1, b1, W2, b2):
    nblk = 16
    rows = BATCH // nblk
    emb_spec = pl.BlockSpec((rows, D), lambda b: (b, 0))
    w_spec = pl.BlockSpec((D, D), lambda b: (0, 0))
    b_spec = pl.BlockSpec((1, D), lambda b: (0, 0))
    out_spec = pl.BlockSpec((rows, 1), lambda b: (b, 0))
    si, sj = pl.pallas_call(
        _mlp_body,
        grid=(nblk,),
        in_specs=[emb_spec, emb_spec, emb_spec, w_spec, b_spec, w_spec, b_spec],
        out_specs=[out_spec, out_spec],
        out_shape=[jax.ShapeDtypeStruct((BATCH, 1), jnp.float32)] * 2,
    )(ue, ie, je, W1, b1.reshape(1, D), W2, b2.reshape(1, D))
    return si.reshape(BATCH), sj.reshape(BATCH)


def kernel(u, i, j, user_emb_w, item_emb_w, W1, b1, W2, b2):
    ue, ie, je = _sc_gather(u, i, j, user_emb_w, item_emb_w)
    return _tc_mlp(ue, ie, je, W1, b1, W2, b2)


# speedup vs baseline: 1.9883x; 1.9883x over previous
import functools

import jax
import jax.numpy as jnp
from jax import lax
from jax.experimental import pallas as pl
from jax.experimental.pallas import tpu as pltpu
from jax.experimental.pallas import tpu_sc as plsc

BATCH = 16384
D = 64
NC = 2
NS = 16
NW = NC * NS
B_PER_W = BATCH // NW
CHUNK = 128
N_CHUNKS = B_PER_W // CHUNK


ROWS_PER_CHUNK = 256


def _sc_gather_body(u_hbm, i_hbm, j_hbm, utab_hbm, itab_hbm,
                    ue_hbm, ie_hbm, je_hbm,
                    idx_u, idx_i, idx_j, buf0, buf1,
                    gsem0, gsem1, wsem0, wsem1):
    wid = lax.axis_index("s") * NC + lax.axis_index("c")
    base = wid * B_PER_W
    pltpu.sync_copy(u_hbm.at[pl.ds(base, B_PER_W)], idx_u)
    pltpu.sync_copy(i_hbm.at[pl.ds(base, B_PER_W)], idx_i)
    pltpu.sync_copy(j_hbm.at[pl.ds(base, B_PER_W)], idx_j)

    bufs = (buf0, buf1)
    gsems = (gsem0, gsem1)
    wsems = (wsem0, wsem1)
    tasks = []
    for idx_ref, out_ref, tab in ((idx_u, ue_hbm, utab_hbm),
                                  (idx_i, ie_hbm, itab_hbm),
                                  (idx_j, je_hbm, itab_hbm)):
        for c in range(B_PER_W // ROWS_PER_CHUNK):
            tasks.append((idx_ref, out_ref, tab, c * ROWS_PER_CHUNK))

    writes = [None] * len(tasks)
    for t, (idx_ref, out_ref, tab, off) in enumerate(tasks):
        b = t % 2
        buf = bufs[b]
        if t >= 2:
            writes[t - 2].wait()

        @pl.loop(0, ROWS_PER_CHUNK // 16)
        def _grp(g, idx_ref=idx_ref, tab=tab, buf=buf, off=off, b=b):
            k0 = g * 16
            v = idx_ref[pl.ds(off + k0, 16)]
            for l in range(16):
                pltpu.async_copy(tab.at[pl.ds(v[l], 1)],
                                 buf.at[pl.ds(k0 + l, 1)], gsems[b])

        pltpu.make_async_copy(tab.at[pl.ds(0, ROWS_PER_CHUNK)], buf,
                              gsems[b]).wait()
        writes[t] = pltpu.async_copy(
            buf, out_ref.at[pl.ds(base + off, ROWS_PER_CHUNK)], wsems[b])
    writes[-2].wait()
    writes[-1].wait()


@jax.jit
def _sc_gather(u, i, j, utab, itab):
    mesh = plsc.VectorSubcoreMesh(core_axis_name="c", subcore_axis_name="s",
                                  num_cores=NC, num_subcores=NS)
    emb = jax.ShapeDtypeStruct((BATCH, D), jnp.float32)
    return pl.kernel(
        _sc_gather_body,
        out_type=(emb, emb, emb),
        mesh=mesh,
        scratch_types=[
            pltpu.VMEM((B_PER_W,), jnp.int32),
            pltpu.VMEM((B_PER_W,), jnp.int32),
            pltpu.VMEM((B_PER_W,), jnp.int32),
            pltpu.VMEM((ROWS_PER_CHUNK, D), jnp.float32),
            pltpu.VMEM((ROWS_PER_CHUNK, D), jnp.float32),
            pltpu.SemaphoreType.DMA,
            pltpu.SemaphoreType.DMA,
            pltpu.SemaphoreType.DMA,
            pltpu.SemaphoreType.DMA,
        ],
    )(u, i, j, utab, itab)


def _mlp_body(ue_ref, ie_ref, je_ref, w1_ref, b1_ref, w2_ref, b2_ref,
              si_ref, sj_ref):
    ue = ue_ref[...]
    h = jnp.dot(ue, w1_ref[...].T, preferred_element_type=jnp.float32)
    h = jnp.maximum(h + b1_ref[...], 0.0)
    h = jnp.dot(h, w2_ref[...].T, preferred_element_type=jnp.float32)
    h = jnp.maximum(h + b2_ref[...], 0.0)
    si_ref[...] = jnp.sum(h * ie_ref[...], axis=1, keepdims=True)
    sj_ref[...] = jnp.sum(h * je_ref[...], axis=1, keepdims=True)


@jax.jit
def _tc_mlp(ue, ie, je, W1, b1, W2, b2):
    nblk = 16
    rows = BATCH // nblk
    emb_spec = pl.BlockSpec((rows, D), lambda b: (b, 0))
    w_spec = pl.BlockSpec((D, D), lambda b: (0, 0))
    b_spec = pl.BlockSpec((1, D), lambda b: (0, 0))
    out_spec = pl.BlockSpec((rows, 1), lambda b: (b, 0))
    si, sj = pl.pallas_call(
        _mlp_body,
        grid=(nblk,),
        in_specs=[emb_spec, emb_spec, emb_spec, w_spec, b_spec, w_spec, b_spec],
        out_specs=[out_spec, out_spec],
        out_shape=[jax.ShapeDtypeStruct((BATCH, 1), jnp.float32)] * 2,
    )(ue, ie, je, W1, b1.reshape(1, D), W2, b2.reshape(1, D))
    return si.reshape(BATCH), sj.reshape(BATCH)


def kernel(u, i, j, user_emb_w, item_emb_w, W1, b1, W2, b2):
    ue, ie, je = _sc_gather(u, i, j, user_emb_w, item_emb_w)
    return _tc_mlp(ue, ie, je, W1, b1, W2, b2)
